# baseline (speedup 1.0000x reference)
import jax
import jax.numpy as jnp
from jax.experimental import pallas as pl
from jax.experimental.pallas import tpu as pltpu

B = 32
T = 4
H = 512
E = 8


def _gates(g):
    i = jax.nn.sigmoid(g[:, :H])
    f = jax.nn.sigmoid(g[:, H:2 * H])
    gg = jnp.tanh(g[:, 2 * H:3 * H])
    o = jax.nn.sigmoid(g[:, 3 * H:])
    return i, f, gg, o


def _cell(gx, gh, b, c):
    i, f, gg, o = _gates(gx + gh + b)
    cn = f * c + i * gg
    hn = o * jnp.tanh(cn)
    return hn, cn


def _cell0(gx, b):
    i, _, gg, o = _gates(gx + b)
    cn = i * gg
    hn = o * jnp.tanh(cn)
    return hn, cn


def _dot(a, b):
    return jax.lax.dot_general(a, b, (((1,), (0,)), ((), ())),
                               preferred_element_type=jnp.float32)


def _moe_kernel(p_ref,
                d_Wi1_h, d_Wh1_h, d_Wi2_h, d_Wh2_h,
                e_Wi1_h, e_Wh1_h, e_Wi2_h, e_Wh2_h, e_Wo_h,
                out_ref,
                d_Wi1, d_Wh1, d_Wi2, d_Wh2,
                e_Wi1, e_Wh1, e_Wi2, e_Wh2, e_Wo,
                *sems):
    hbm = (d_Wi1_h, d_Wi2_h, e_Wi1_h, e_Wi2_h, e_Wo_h, e_Wh1_h, e_Wh2_h,
           d_Wh1_h, d_Wh2_h)
    vmem = (d_Wi1, d_Wi2, e_Wi1, e_Wi2, e_Wo, e_Wh1, e_Wh2, d_Wh1, d_Wh2)
    copies = {}
    for src, dst, sem in zip(hbm, vmem, sems):
        c = pltpu.make_async_copy(src, dst, sem)
        c.start()
        copies[id(dst)] = c

    waited = set()

    def w(ref):
        if id(ref) not in waited:
            copies[id(ref)].wait()
            waited.add(id(ref))
        return ref[...]

    x0 = p_ref[0:B, 0:H]
    d_b1 = p_ref[B:B + 1, :]
    d_b2 = p_ref[B + 1:B + 2, :]
    e_b1 = p_ref[B + 2:B + 3, :]
    e_b2 = p_ref[B + 3:B + 4, :]
    e_bo = p_ref[B + 4:B + 5, 0:H]
    g_b = p_ref[B + 5:B + 6, 0:E]
    g_WT = p_ref[B + 6:B + 6 + E, 0:H]

    lane = jax.lax.broadcasted_iota(jnp.int32, (B, E), 1)

    def gate_weights(d_c2):
        logits = jax.lax.dot_general(
            d_c2, g_WT, (((1,), (1,)), ((), ())),
            preferred_element_type=jnp.float32) + g_b
        m = jnp.max(logits, axis=1, keepdims=True)
        ex = jnp.exp(logits - m)
        p = ex / jnp.sum(ex, axis=1, keepdims=True)
        m1 = jnp.max(p, axis=1, keepdims=True)
        i1 = jnp.min(jnp.where(p == m1, lane, E), axis=1, keepdims=True)
        p2 = jnp.where(lane == i1, -1.0, p)
        m2 = jnp.max(p2, axis=1, keepdims=True)
        i2 = jnp.min(jnp.where(p2 == m2, lane, E), axis=1, keepdims=True)
        w_lo = jnp.where(i1 < i2, m1, m2)
        w_hi = jnp.where(i1 < i2, m2, m1)
        return w_lo, w_hi

    d_h1, d_c1 = _cell0(_dot(x0, w(d_Wi1)), d_b1)
    d_h2, d_c2 = _cell0(_dot(d_h1, w(d_Wi2)), d_b2)
    w_lo, w_hi = gate_weights(d_c2)

    xw = _dot(x0, w(e_Wi1))
    h1a, c1a = _cell0(xw, e_b1)
    h2a, c2a = _cell0(_dot(h1a, w(e_Wi2)), e_b2)
    out_a = _dot(h2a, w(e_Wo)) + e_bo
    h1b, c1b = _cell(xw, _dot(h1a, w(e_Wh1)), e_b1, c1a)
    h2b, c2b = _cell(_dot(h1b, e_Wi2[...]), _dot(h2a, w(e_Wh2)), e_b2, c2a)
    out_b = _dot(h2b, e_Wo[...]) + e_bo
    e_h1, e_c1, e_h2, e_c2 = h1b, c1b, h2b, c2b

    o = w_lo * out_a + w_hi * out_b
    out_ref[:, 0:H] = o

    for t in range(1, T):
        xt = o
        d_h1, d_c1 = _cell(_dot(xt, d_Wi1[...]),
                           _dot(d_h1, w(d_Wh1)), d_b1, d_c1)
        d_h2, d_c2 = _cell(_dot(d_h1, d_Wi2[...]),
                           _dot(d_h2, w(d_Wh2)), d_b2, d_c2)
        w_lo, w_hi = gate_weights(d_c2)

        xw = _dot(xt, e_Wi1[...])
        h1a, c1a = _cell(xw, _dot(e_h1, e_Wh1[...]), e_b1, e_c1)
        h2a, c2a = _cell(_dot(h1a, e_Wi2[...]),
                         _dot(e_h2, e_Wh2[...]), e_b2, e_c2)
        out_a = _dot(h2a, e_Wo[...]) + e_bo
        h1b, c1b = _cell(xw, _dot(h1a, e_Wh1[...]), e_b1, c1a)
        h2b, c2b = _cell(_dot(h1b, e_Wi2[...]),
                         _dot(h2a, e_Wh2[...]), e_b2, c2a)
        out_b = _dot(h2b, e_Wo[...]) + e_bo
        e_h1, e_c1, e_h2, e_c2 = h1b, c1b, h2b, c2b

        o = w_lo * out_a + w_hi * out_b
        out_ref[:, t * H:(t + 1) * H] = o


def kernel(x, d_Wi1, d_Wh1, d_b1, d_Wi2, d_Wh2, d_b2, d_Wo, d_bo,
           g_W, g_b,
           e_Wi1, e_Wh1, e_b1, e_Wi2, e_Wh2, e_b2, e_Wo, e_bo):
    del d_Wo, d_bo
    p = jnp.zeros((B + 6 + E, 4 * H), dtype=jnp.float32)
    p = p.at[0:B, 0:H].set(x[:, 0, :])
    p = p.at[B, :].set(d_b1)
    p = p.at[B + 1, :].set(d_b2)
    p = p.at[B + 2, :].set(e_b1)
    p = p.at[B + 3, :].set(e_b2)
    p = p.at[B + 4, 0:H].set(e_bo)
    p = p.at[B + 5, 0:E].set(g_b)
    p = p.at[B + 6:B + 6 + E, 0:H].set(g_W.T)

    n_big = 9
    big_shapes = [(H, 4 * H)] * 8 + [(H, H)]
    out = pl.pallas_call(
        _moe_kernel,
        out_shape=jax.ShapeDtypeStruct((B, T * H), jnp.float32),
        in_specs=(
            [pl.BlockSpec(memory_space=pltpu.MemorySpace.VMEM)]
            + [pl.BlockSpec(memory_space=pltpu.MemorySpace.HBM)] * n_big),
        out_specs=pl.BlockSpec(memory_space=pltpu.MemorySpace.VMEM),
        scratch_shapes=(
            [pltpu.VMEM(s, jnp.float32) for s in big_shapes]
            + [pltpu.SemaphoreType.DMA] * n_big),
    )(p, d_Wi1, d_Wh1, d_Wi2, d_Wh2, e_Wi1, e_Wh1, e_Wi2, e_Wh2, e_Wo)
    return out.reshape(B, T, H)


# speedup vs baseline: 1.1047x; 1.1047x over previous
import jax
import jax.numpy as jnp
from jax.experimental import pallas as pl
from jax.experimental.pallas import tpu as pltpu

B = 32
T = 4
H = 512
E = 8
NCHUNK = 1


def _gates(g):
    i = jax.nn.sigmoid(g[:, :H])
    f = jax.nn.sigmoid(g[:, H:2 * H])
    gg = jnp.tanh(g[:, 2 * H:3 * H])
    o = jax.nn.sigmoid(g[:, 3 * H:])
    return i, f, gg, o


def _cell(gx, gh, b, c):
    i, f, gg, o = _gates(gx + gh + b)
    cn = f * c + i * gg
    hn = o * jnp.tanh(cn)
    return hn, cn


def _cell0(gx, b):
    i, _, gg, o = _gates(gx + b)
    cn = i * gg
    hn = o * jnp.tanh(cn)
    return hn, cn


def _dot(a, b):
    return jax.lax.dot_general(a, b, (((1,), (0,)), ((), ())),
                               preferred_element_type=jnp.float32)


def _moe_kernel(x0_ref, d_b1_ref, d_b2_ref, g_W_ref, g_b_ref,
                e_b1_ref, e_b2_ref, e_bo_ref,
                d_Wi1_h, d_Wh1_h, d_Wi2_h, d_Wh2_h,
                e_Wi1_h, e_Wh1_h, e_Wi2_h, e_Wh2_h, e_Wo_h,
                out_ref,
                d_Wi1, d_Wh1, d_Wi2, d_Wh2,
                e_Wi1, e_Wh1, e_Wi2, e_Wh2, e_Wo,
                *sems):
    hbm = (d_Wi1_h, d_Wi2_h, e_Wi1_h, e_Wi2_h, e_Wo_h, e_Wh1_h, e_Wh2_h,
           d_Wh1_h, d_Wh2_h)
    vmem = (d_Wi1, d_Wi2, e_Wi1, e_Wi2, e_Wo, e_Wh1, e_Wh2, d_Wh1, d_Wh2)
    copies = {}
    si = iter(sems)
    for src, dst in zip(hbm, vmem):
        cs = []
        for k in range(NCHUNK):
            sl = pl.ds(k * (512 // NCHUNK), 512 // NCHUNK)
            c = pltpu.make_async_copy(src.at[sl, :], dst.at[sl, :], next(si))
            c.start()
            cs.append(c)
        copies[id(dst)] = cs

    waited = set()

    def w(ref):
        if id(ref) not in waited:
            for c in copies[id(ref)]:
                c.wait()
            waited.add(id(ref))
        return ref[...]

    x0 = x0_ref[...]
    d_b1 = d_b1_ref[...]
    d_b2 = d_b2_ref[...]
    e_b1 = e_b1_ref[...]
    e_b2 = e_b2_ref[...]
    e_bo = e_bo_ref[...]
    g_b = g_b_ref[...]

    lane = jax.lax.broadcasted_iota(jnp.int32, (B, E), 1)

    def gate_weights(d_c2):
        logits = _dot(d_c2, g_W_ref[...]) + g_b
        m = jnp.max(logits, axis=1, keepdims=True)
        ex = jnp.exp(logits - m)
        p = ex / jnp.sum(ex, axis=1, keepdims=True)
        m1 = jnp.max(p, axis=1, keepdims=True)
        i1 = jnp.min(jnp.where(p == m1, lane, E), axis=1, keepdims=True)
        p2 = jnp.where(lane == i1, -1.0, p)
        m2 = jnp.max(p2, axis=1, keepdims=True)
        i2 = jnp.min(jnp.where(p2 == m2, lane, E), axis=1, keepdims=True)
        w_lo = jnp.where(i1 < i2, m1, m2)
        w_hi = jnp.where(i1 < i2, m2, m1)
        return w_lo, w_hi

    d_h1, d_c1 = _cell0(_dot(x0, w(d_Wi1)), d_b1)
    d_h2, d_c2 = _cell0(_dot(d_h1, w(d_Wi2)), d_b2)
    w_lo, w_hi = gate_weights(d_c2)

    xw = _dot(x0, w(e_Wi1))
    h1a, c1a = _cell0(xw, e_b1)
    h2a, c2a = _cell0(_dot(h1a, w(e_Wi2)), e_b2)
    out_a = _dot(h2a, w(e_Wo)) + e_bo
    h1b, c1b = _cell(xw, _dot(h1a, w(e_Wh1)), e_b1, c1a)
    h2b, c2b = _cell(_dot(h1b, e_Wi2[...]), _dot(h2a, w(e_Wh2)), e_b2, c2a)
    out_b = _dot(h2b, e_Wo[...]) + e_bo
    e_h1, e_c1, e_h2, e_c2 = h1b, c1b, h2b, c2b

    o = w_lo * out_a + w_hi * out_b
    out_ref[:, 0:H] = o

    for t in range(1, T):
        xt = o
        xw = _dot(xt, e_Wi1[...])
        h1a, c1a = _cell(xw, _dot(e_h1, e_Wh1[...]), e_b1, e_c1)
        h2a, c2a = _cell(_dot(h1a, e_Wi2[...]),
                         _dot(e_h2, e_Wh2[...]), e_b2, e_c2)
        out_a = _dot(h2a, e_Wo[...]) + e_bo
        h1b, c1b = _cell(xw, _dot(h1a, e_Wh1[...]), e_b1, c1a)
        h2b, c2b = _cell(_dot(h1b, e_Wi2[...]),
                         _dot(h2a, e_Wh2[...]), e_b2, c2a)
        out_b = _dot(h2b, e_Wo[...]) + e_bo
        e_h1, e_c1, e_h2, e_c2 = h1b, c1b, h2b, c2b

        d_h1, d_c1 = _cell(_dot(xt, d_Wi1[...]),
                           _dot(d_h1, w(d_Wh1)), d_b1, d_c1)
        d_h2, d_c2 = _cell(_dot(d_h1, d_Wi2[...]),
                           _dot(d_h2, w(d_Wh2)), d_b2, d_c2)
        w_lo, w_hi = gate_weights(d_c2)

        o = w_lo * out_a + w_hi * out_b
        out_ref[:, t * H:(t + 1) * H] = o


def kernel(x, d_Wi1, d_Wh1, d_b1, d_Wi2, d_Wh2, d_b2, d_Wo, d_bo,
           g_W, g_b,
           e_Wi1, e_Wh1, e_b1, e_Wi2, e_Wh2, e_b2, e_Wo, e_bo):
    del d_Wo, d_bo
    x0 = x[:, 0, :]
    n_small = 8
    n_big = 9
    big_shapes = [(512, 2048)] * 8 + [(512, 512)]
    out = pl.pallas_call(
        _moe_kernel,
        out_shape=jax.ShapeDtypeStruct((B, T * H), jnp.float32),
        in_specs=(
            [pl.BlockSpec(memory_space=pltpu.MemorySpace.VMEM)] * n_small
            + [pl.BlockSpec(memory_space=pltpu.MemorySpace.HBM)] * n_big),
        out_specs=pl.BlockSpec(memory_space=pltpu.MemorySpace.VMEM),
        scratch_shapes=(
            [pltpu.VMEM(s, jnp.float32) for s in big_shapes]
            + [pltpu.SemaphoreType.DMA] * (n_big * NCHUNK)),
    )(x0, d_b1.reshape(1, -1), d_b2.reshape(1, -1), g_W, g_b.reshape(1, -1),
      e_b1.reshape(1, -1), e_b2.reshape(1, -1), e_bo.reshape(1, -1),
      d_Wi1, d_Wh1, d_Wi2, d_Wh2, e_Wi1, e_Wh1, e_Wi2, e_Wh2, e_Wo)
    return out.reshape(B, T, H)


# speedup vs baseline: 1.1881x; 1.0756x over previous
import jax
import jax.numpy as jnp
from jax.experimental import pallas as pl
from jax.experimental.pallas import tpu as pltpu

B = 32
T = 4
H = 512
E = 8


def _gates(g):
    i = jax.nn.sigmoid(g[:, :H])
    f = jax.nn.sigmoid(g[:, H:2 * H])
    gg = jnp.tanh(g[:, 2 * H:3 * H])
    o = jax.nn.sigmoid(g[:, 3 * H:])
    return i, f, gg, o


def _cell(gx, gh, b, c):
    i, f, gg, o = _gates(gx + gh + b)
    cn = f * c + i * gg
    hn = o * jnp.tanh(cn)
    return hn, cn


def _cell0(gx, b):
    i, _, gg, o = _gates(gx + b)
    cn = i * gg
    hn = o * jnp.tanh(cn)
    return hn, cn


def _dot(a, b):
    return jax.lax.dot_general(a, b, (((1,), (0,)), ((), ())),
                               preferred_element_type=jnp.float32)


def _moe_kernel(x0_ref, d_b1_ref, d_b2_ref, g_W_ref, g_b_ref,
                e_b1_ref, e_b2_ref, e_bo_ref,
                d_Wi1_h, d_Wh1_h, d_Wi2_h, d_Wh2_h,
                e_Wi1_h, e_Wh1_h, e_Wi2_h, e_Wh2_h, e_Wo_h,
                out_ref,
                d_Wi1, d_Wh1, d_Wi2, d_Wh2,
                e_Wi1, e_Wh1, e_Wi2, e_Wh2, e_Wo,
                *sems):
    hbm = (d_Wi1_h, d_Wi2_h, e_Wi1_h, e_Wi2_h, e_Wo_h, e_Wh1_h, e_Wh2_h,
           d_Wh1_h, d_Wh2_h)
    vmem = (d_Wi1, d_Wi2, e_Wi1, e_Wi2, e_Wo, e_Wh1, e_Wh2, d_Wh1, d_Wh2)
    copies = {}
    for src, dst, sem in zip(hbm, vmem, sems):
        c = pltpu.make_async_copy(src, dst, sem)
        c.start()
        copies[id(dst)] = c

    waited = set()

    def w(ref):
        if id(ref) not in waited:
            copies[id(ref)].wait()
            waited.add(id(ref))
        return ref[...]

    x0 = x0_ref[...]
    d_b1 = d_b1_ref[...]
    d_b2 = d_b2_ref[...]
    e_b1 = e_b1_ref[...]
    e_b2 = e_b2_ref[...]
    e_bo = e_bo_ref[...]
    g_b = g_b_ref[...]

    lane = jax.lax.broadcasted_iota(jnp.int32, (B, E), 1)

    def gate_weights(d_c2):
        logits = _dot(d_c2, g_W_ref[...]) + g_b
        m = jnp.max(logits, axis=1, keepdims=True)
        ex = jnp.exp(logits - m)
        p = ex / jnp.sum(ex, axis=1, keepdims=True)
        m1 = jnp.max(p, axis=1, keepdims=True)
        i1 = jnp.min(jnp.where(p == m1, lane, E), axis=1, keepdims=True)
        p2 = jnp.where(lane == i1, -1.0, p)
        m2 = jnp.max(p2, axis=1, keepdims=True)
        i2 = jnp.min(jnp.where(p2 == m2, lane, E), axis=1, keepdims=True)
        w_lo = jnp.where(i1 < i2, m1, m2)
        w_hi = jnp.where(i1 < i2, m2, m1)
        return w_lo, w_hi

    d_h1, d_c1 = _cell0(_dot(x0, w(d_Wi1)), d_b1)
    d_h2, d_c2 = _cell0(_dot(d_h1, w(d_Wi2)), d_b2)
    w_lo, w_hi = gate_weights(d_c2)

    xw = _dot(x0, w(e_Wi1))
    h1a, c1a = _cell0(xw, e_b1)
    h2a, c2a = _cell0(_dot(h1a, w(e_Wi2)), e_b2)
    out_a = _dot(h2a, w(e_Wo)) + e_bo
    h1b, c1b = _cell(xw, _dot(h1a, w(e_Wh1)), e_b1, c1a)
    h2b, c2b = _cell(_dot(h1b, e_Wi2[...]), _dot(h2a, w(e_Wh2)), e_b2, c2a)
    out_b = _dot(h2b, e_Wo[...]) + e_bo
    e_h1, e_c1, e_h2, e_c2 = h1b, c1b, h2b, c2b

    o = w_lo * out_a + w_hi * out_b
    out_ref[:, 0:H] = o

    for t in range(1, T):
        xt = o
        d_h1, d_c1 = _cell(_dot(xt, d_Wi1[...]),
                           _dot(d_h1, w(d_Wh1)), d_b1, d_c1)
        d_h2, d_c2 = _cell(_dot(d_h1, d_Wi2[...]),
                           _dot(d_h2, w(d_Wh2)), d_b2, d_c2)
        w_lo, w_hi = gate_weights(d_c2)

        xw = _dot(xt, e_Wi1[...])
        h1a, c1a = _cell(xw, _dot(e_h1, e_Wh1[...]), e_b1, e_c1)
        h2a, c2a = _cell(_dot(h1a, e_Wi2[...]),
                         _dot(e_h2, e_Wh2[...]), e_b2, e_c2)
        out_a = _dot(h2a, e_Wo[...]) + e_bo
        h1b, c1b = _cell(xw, _dot(h1a, e_Wh1[...]), e_b1, c1a)
        h2b, c2b = _cell(_dot(h1b, e_Wi2[...]),
                         _dot(h2a, e_Wh2[...]), e_b2, c2a)
        out_b = _dot(h2b, e_Wo[...]) + e_bo
        e_h1, e_c1, e_h2, e_c2 = h1b, c1b, h2b, c2b

        o = w_lo * out_a + w_hi * out_b
        out_ref[:, t * H:(t + 1) * H] = o


def kernel(x, d_Wi1, d_Wh1, d_b1, d_Wi2, d_Wh2, d_b2, d_Wo, d_bo,
           g_W, g_b,
           e_Wi1, e_Wh1, e_b1, e_Wi2, e_Wh2, e_b2, e_Wo, e_bo):
    del d_Wo, d_bo
    x0 = x[:, 0, :]
    n_small = 8
    n_big = 9
    big_shapes = [(512, 2048)] * 8 + [(512, 512)]
    out = pl.pallas_call(
        _moe_kernel,
        out_shape=jax.ShapeDtypeStruct((B, T * H), jnp.float32),
        in_specs=(
            [pl.BlockSpec(memory_space=pltpu.MemorySpace.VMEM)] * n_small
            + [pl.BlockSpec(memory_space=pltpu.MemorySpace.HBM)] * n_big),
        out_specs=pl.BlockSpec(memory_space=pltpu.MemorySpace.VMEM),
        scratch_shapes=(
            [pltpu.VMEM(s, jnp.float32) for s in big_shapes]
            + [pltpu.SemaphoreType.DMA] * n_big),
    )(x0, d_b1.reshape(1, -1), d_b2.reshape(1, -1), g_W, g_b.reshape(1, -1),
      e_b1.reshape(1, -1), e_b2.reshape(1, -1), e_bo.reshape(1, -1),
      d_Wi1, d_Wh1, d_Wi2, d_Wh2, e_Wi1, e_Wh1, e_Wi2, e_Wh2, e_Wo)
    return out.reshape(B, T, H)


# speedup vs baseline: 1.3322x; 1.1213x over previous
import jax
import jax.numpy as jnp
from jax.experimental import pallas as pl
from jax.experimental.pallas import tpu as pltpu

B = 32
T = 4
H = 512
E = 8


def _gates(g):
    i = jax.nn.sigmoid(g[:, :H])
    f = jax.nn.sigmoid(g[:, H:2 * H])
    gg = jnp.tanh(g[:, 2 * H:3 * H])
    o = jax.nn.sigmoid(g[:, 3 * H:])
    return i, f, gg, o


def _cell(gx, gh, b, c):
    i, f, gg, o = _gates(gx + gh + b)
    cn = f * c + i * gg
    hn = o * jnp.tanh(cn)
    return hn, cn


def _cell0(gx, b):
    i, _, gg, o = _gates(gx + b)
    cn = i * gg
    hn = o * jnp.tanh(cn)
    return hn, cn


def _dot(a, b):
    return jax.lax.dot_general(a, b, (((1,), (0,)), ((), ())),
                               preferred_element_type=jnp.float32)


def _moe_kernel(x_ref, d_b1_ref, d_b2_ref, g_W_ref, g_b_ref,
                e_b1_ref, e_b2_ref, e_bo_ref,
                d_Wi1_h, d_Wh1_h, d_Wi2_h, d_Wh2_h,
                e_Wi1_h, e_Wh1_h, e_Wi2_h, e_Wh2_h, e_Wo_h,
                out_ref,
                d_Wi1, d_Wh1, d_Wi2, d_Wh2,
                e_Wi1, e_Wh1, e_Wi2, e_Wh2, e_Wo,
                *sems):
    hbm = (d_Wi1_h, d_Wi2_h, e_Wi1_h, e_Wi2_h, e_Wo_h, e_Wh1_h, e_Wh2_h,
           d_Wh1_h, d_Wh2_h)
    vmem = (d_Wi1, d_Wi2, e_Wi1, e_Wi2, e_Wo, e_Wh1, e_Wh2, d_Wh1, d_Wh2)
    copies = {}
    for src, dst, sem in zip(hbm, vmem, sems):
        c = pltpu.make_async_copy(src, dst, sem)
        c.start()
        copies[id(dst)] = c

    waited = set()

    def w(ref):
        if id(ref) not in waited:
            copies[id(ref)].wait()
            waited.add(id(ref))
        return ref[...]

    x0 = x_ref[:, 0, :]
    d_b1 = d_b1_ref[...]
    d_b2 = d_b2_ref[...]
    e_b1 = e_b1_ref[...]
    e_b2 = e_b2_ref[...]
    e_bo = e_bo_ref[...]
    g_b = g_b_ref[...]

    lane = jax.lax.broadcasted_iota(jnp.int32, (B, E), 1)

    def gate_weights(d_c2):
        logits = _dot(d_c2, g_W_ref[...]) + g_b
        m = jnp.max(logits, axis=1, keepdims=True)
        ex = jnp.exp(logits - m)
        p = ex / jnp.sum(ex, axis=1, keepdims=True)
        m1 = jnp.max(p, axis=1, keepdims=True)
        i1 = jnp.min(jnp.where(p == m1, lane, E), axis=1, keepdims=True)
        p2 = jnp.where(lane == i1, -1.0, p)
        m2 = jnp.max(p2, axis=1, keepdims=True)
        i2 = jnp.min(jnp.where(p2 == m2, lane, E), axis=1, keepdims=True)
        w_lo = jnp.where(i1 < i2, m1, m2)
        w_hi = jnp.where(i1 < i2, m2, m1)
        return w_lo, w_hi

    d_h1, d_c1 = _cell0(_dot(x0, w(d_Wi1)), d_b1)
    d_h2, d_c2 = _cell0(_dot(d_h1, w(d_Wi2)), d_b2)
    w_lo, w_hi = gate_weights(d_c2)

    xw = _dot(x0, w(e_Wi1))
    h1a, c1a = _cell0(xw, e_b1)
    h2a, c2a = _cell0(_dot(h1a, w(e_Wi2)), e_b2)
    out_a = _dot(h2a, w(e_Wo)) + e_bo
    h1b, c1b = _cell(xw, _dot(h1a, w(e_Wh1)), e_b1, c1a)
    h2b, c2b = _cell(_dot(h1b, e_Wi2[...]), _dot(h2a, w(e_Wh2)), e_b2, c2a)
    out_b = _dot(h2b, e_Wo[...]) + e_bo
    e_h1, e_c1, e_h2, e_c2 = h1b, c1b, h2b, c2b

    o = w_lo * out_a + w_hi * out_b
    out_ref[:, 0, :] = o

    for t in range(1, T):
        xt = o
        d_h1, d_c1 = _cell(_dot(xt, d_Wi1[...]),
                           _dot(d_h1, w(d_Wh1)), d_b1, d_c1)
        d_h2, d_c2 = _cell(_dot(d_h1, d_Wi2[...]),
                           _dot(d_h2, w(d_Wh2)), d_b2, d_c2)
        w_lo, w_hi = gate_weights(d_c2)

        xw = _dot(xt, e_Wi1[...])
        h1a, c1a = _cell(xw, _dot(e_h1, e_Wh1[...]), e_b1, e_c1)
        h2a, c2a = _cell(_dot(h1a, e_Wi2[...]),
                         _dot(e_h2, e_Wh2[...]), e_b2, e_c2)
        out_a = _dot(h2a, e_Wo[...]) + e_bo
        h1b, c1b = _cell(xw, _dot(h1a, e_Wh1[...]), e_b1, c1a)
        h2b, c2b = _cell(_dot(h1b, e_Wi2[...]),
                         _dot(h2a, e_Wh2[...]), e_b2, c2a)
        out_b = _dot(h2b, e_Wo[...]) + e_bo
        e_h1, e_c1, e_h2, e_c2 = h1b, c1b, h2b, c2b

        o = w_lo * out_a + w_hi * out_b
        out_ref[:, t, :] = o


def kernel(x, d_Wi1, d_Wh1, d_b1, d_Wi2, d_Wh2, d_b2, d_Wo, d_bo,
           g_W, g_b,
           e_Wi1, e_Wh1, e_b1, e_Wi2, e_Wh2, e_b2, e_Wo, e_bo):
    del d_Wo, d_bo
    n_small = 8
    n_big = 9
    big_shapes = [(512, 2048)] * 8 + [(512, 512)]
    out = pl.pallas_call(
        _moe_kernel,
        out_shape=jax.ShapeDtypeStruct((B, T, H), jnp.float32),
        in_specs=(
            [pl.BlockSpec(memory_space=pltpu.MemorySpace.VMEM)] * n_small
            + [pl.BlockSpec(memory_space=pltpu.MemorySpace.HBM)] * n_big),
        out_specs=pl.BlockSpec(memory_space=pltpu.MemorySpace.VMEM),
        scratch_shapes=(
            [pltpu.VMEM(s, jnp.float32) for s in big_shapes]
            + [pltpu.SemaphoreType.DMA] * n_big),
    )(x, d_b1.reshape(1, -1), d_b2.reshape(1, -1), g_W, g_b.reshape(1, -1),
      e_b1.reshape(1, -1), e_b2.reshape(1, -1), e_bo.reshape(1, -1),
      d_Wi1, d_Wh1, d_Wi2, d_Wh2, e_Wi1, e_Wh1, e_Wi2, e_Wh2, e_Wo)
    return out


# speedup vs baseline: 1.3405x; 1.0062x over previous
import jax
import jax.numpy as jnp
from jax.experimental import pallas as pl
from jax.experimental.pallas import tpu as pltpu

B = 32
T = 4
H = 512
E = 8


def _gates(g):
    i = jax.nn.sigmoid(g[:, :H])
    f = jax.nn.sigmoid(g[:, H:2 * H])
    gg = jnp.tanh(g[:, 2 * H:3 * H])
    o = jax.nn.sigmoid(g[:, 3 * H:])
    return i, f, gg, o


def _cell(gx, gh, b, c):
    i, f, gg, o = _gates(gx + gh + b)
    cn = f * c + i * gg
    hn = o * jnp.tanh(cn)
    return hn, cn


def _cell0(gx, b):
    i, _, gg, o = _gates(gx + b)
    cn = i * gg
    hn = o * jnp.tanh(cn)
    return hn, cn


def _dot(a, b):
    return jax.lax.dot_general(a, b, (((1,), (0,)), ((), ())),
                               preferred_element_type=jnp.float32)


def _moe_kernel(x_ref, d_b1_ref, d_b2_ref, g_W_ref, g_b_ref,
                e_b1_ref, e_b2_ref, e_bo_ref,
                d_Wi1_h, d_Wh1_h, d_Wi2_h, d_Wh2_h,
                e_Wi1_h, e_Wh1_h, e_Wi2_h, e_Wh2_h, e_Wo_h,
                out_ref,
                x0_s,
                d_Wi1, d_Wh1, d_Wi2, d_Wh2,
                e_Wi1, e_Wh1, e_Wi2, e_Wh2, e_Wo,
                *sems):
    hbm = (d_Wi1_h, d_Wi2_h, e_Wi1_h, e_Wi2_h, e_Wo_h, e_Wh1_h, e_Wh2_h,
           d_Wh1_h, d_Wh2_h)
    vmem = (d_Wi1, d_Wi2, e_Wi1, e_Wi2, e_Wo, e_Wh1, e_Wh2, d_Wh1, d_Wh2)
    copies = {}
    cx = pltpu.make_async_copy(x_ref.at[:, 0, :], x0_s, sems[-1])
    cx.start()
    for src, dst, sem in zip(hbm, vmem, sems):
        c = pltpu.make_async_copy(src, dst, sem)
        c.start()
        copies[id(dst)] = c

    waited = set()

    def w(ref):
        if id(ref) not in waited:
            copies[id(ref)].wait()
            waited.add(id(ref))
        return ref[...]

    cx.wait()
    x0 = x0_s[...]
    d_b1 = d_b1_ref[...]
    d_b2 = d_b2_ref[...]
    e_b1 = e_b1_ref[...]
    e_b2 = e_b2_ref[...]
    e_bo = e_bo_ref[...]
    g_b = g_b_ref[...]

    lane = jax.lax.broadcasted_iota(jnp.int32, (B, E), 1)

    def gate_weights(d_c2):
        logits = _dot(d_c2, g_W_ref[...]) + g_b
        m = jnp.max(logits, axis=1, keepdims=True)
        ex = jnp.exp(logits - m)
        p = ex / jnp.sum(ex, axis=1, keepdims=True)
        m1 = jnp.max(p, axis=1, keepdims=True)
        i1 = jnp.min(jnp.where(p == m1, lane, E), axis=1, keepdims=True)
        p2 = jnp.where(lane == i1, -1.0, p)
        m2 = jnp.max(p2, axis=1, keepdims=True)
        i2 = jnp.min(jnp.where(p2 == m2, lane, E), axis=1, keepdims=True)
        w_lo = jnp.where(i1 < i2, m1, m2)
        w_hi = jnp.where(i1 < i2, m2, m1)
        return w_lo, w_hi

    d_h1, d_c1 = _cell0(_dot(x0, w(d_Wi1)), d_b1)
    d_h2, d_c2 = _cell0(_dot(d_h1, w(d_Wi2)), d_b2)
    w_lo, w_hi = gate_weights(d_c2)

    xw = _dot(x0, w(e_Wi1))
    h1a, c1a = _cell0(xw, e_b1)
    h2a, c2a = _cell0(_dot(h1a, w(e_Wi2)), e_b2)
    out_a = _dot(h2a, w(e_Wo)) + e_bo
    h1b, c1b = _cell(xw, _dot(h1a, w(e_Wh1)), e_b1, c1a)
    h2b, c2b = _cell(_dot(h1b, e_Wi2[...]), _dot(h2a, w(e_Wh2)), e_b2, c2a)
    out_b = _dot(h2b, e_Wo[...]) + e_bo
    e_h1, e_c1, e_h2, e_c2 = h1b, c1b, h2b, c2b

    o = w_lo * out_a + w_hi * out_b
    out_ref[:, 0, :] = o

    for t in range(1, T):
        xt = o
        d_h1, d_c1 = _cell(_dot(xt, d_Wi1[...]),
                           _dot(d_h1, w(d_Wh1)), d_b1, d_c1)
        d_h2, d_c2 = _cell(_dot(d_h1, d_Wi2[...]),
                           _dot(d_h2, w(d_Wh2)), d_b2, d_c2)
        w_lo, w_hi = gate_weights(d_c2)

        xw = _dot(xt, e_Wi1[...])
        h1a, c1a = _cell(xw, _dot(e_h1, e_Wh1[...]), e_b1, e_c1)
        h2a, c2a = _cell(_dot(h1a, e_Wi2[...]),
                         _dot(e_h2, e_Wh2[...]), e_b2, e_c2)
        out_a = _dot(h2a, e_Wo[...]) + e_bo
        h1b, c1b = _cell(xw, _dot(h1a, e_Wh1[...]), e_b1, c1a)
        h2b, c2b = _cell(_dot(h1b, e_Wi2[...]),
                         _dot(h2a, e_Wh2[...]), e_b2, c2a)
        out_b = _dot(h2b, e_Wo[...]) + e_bo
        e_h1, e_c1, e_h2, e_c2 = h1b, c1b, h2b, c2b

        o = w_lo * out_a + w_hi * out_b
        out_ref[:, t, :] = o


def kernel(x, d_Wi1, d_Wh1, d_b1, d_Wi2, d_Wh2, d_b2, d_Wo, d_bo,
           g_W, g_b,
           e_Wi1, e_Wh1, e_b1, e_Wi2, e_Wh2, e_b2, e_Wo, e_bo):
    del d_Wo, d_bo
    n_small = 8
    n_big = 9
    big_shapes = [(512, 2048)] * 8 + [(512, 512)]
    out = pl.pallas_call(
        _moe_kernel,
        out_shape=jax.ShapeDtypeStruct((B, T, H), jnp.float32),
        in_specs=(
            [pl.BlockSpec(memory_space=pltpu.MemorySpace.HBM)]
            + [pl.BlockSpec(memory_space=pltpu.MemorySpace.VMEM)] * (n_small - 1)
            + [pl.BlockSpec(memory_space=pltpu.MemorySpace.HBM)] * n_big),
        out_specs=pl.BlockSpec(memory_space=pltpu.MemorySpace.VMEM),
        scratch_shapes=(
            [pltpu.VMEM((B, H), jnp.float32)]
            + [pltpu.VMEM(s, jnp.float32) for s in big_shapes]
            + [pltpu.SemaphoreType.DMA] * (n_big + 1)),
    )(x, d_b1.reshape(1, -1), d_b2.reshape(1, -1), g_W, g_b.reshape(1, -1),
      e_b1.reshape(1, -1), e_b2.reshape(1, -1), e_bo.reshape(1, -1),
      d_Wi1, d_Wh1, d_Wi2, d_Wh2, e_Wi1, e_Wh1, e_Wi2, e_Wh2, e_Wo)
    return out
